# baseline (speedup 1.0000x reference)
import functools

import jax
import jax.numpy as jnp
from jax import lax
from jax.experimental import pallas as pl
from jax.experimental.pallas import tpu as pltpu
from jax.experimental.pallas import tpu_sc as plsc

B = 4096
S = 200
HID = 64
NC, NS = 2, 16
NW = NC * NS
UNIT = 256
CH = 128
UNITS_PER_S = B // UNIT
NUNIT = S * UNITS_PER_S
UPW = NUNIT // NW
NT = UPW // 4
SCALE = 8.0
TB_C, TB_L = 3, 136
ST_HT, ST_C, ST_R = 8 * TB_C * TB_L, 8 * TB_L, TB_L


def _gather_desc(table_hbm, idxb, gbuf, gsem):
    return pltpu.make_async_copy(
        table_hbm.at[idxb.at[pl.ds(0, CH)]], gbuf, gsem)


def _out_descs(tbuf, o_hbm, osem):
    return [
        pltpu.make_async_copy(
            tbuf.at[ht, pl.ds(0, 2), :, pl.ds(0, CH)],
            o_hbm.at[0, ht, pl.ds(0, 2)],
            osem,
        )
        for ht in range(8)
    ]


def _emb_body(xT_hbm, table_hbm, o_hbm, idxb, gbuf, tbuf, isem, gsem, osem):
    wid = lax.axis_index("s") * NC + lax.axis_index("c")
    u0 = wid * UPW

    iota = lax.iota(jnp.int32, 16)
    z16 = jnp.zeros((16,), jnp.int32)
    srcc, dstc = [], []
    for dd in range(16):
        hmod = lax.bitwise_and(iota + dd, 15)
        srcc.append(iota * HID + hmod)
        dstc.append(
            ST_HT * lax.shift_right_logical(hmod, 3)
            + ST_R * lax.bitwise_and(hmod, 7)
            + iota
        )

    def fire_idx(u_local, buf):
        uu = u0 + u_local
        s = uu // UNITS_PER_S
        q = uu % UNITS_PER_S
        pltpu.async_copy(
            xT_hbm.at[s, pl.ds(q * UNIT, UNIT)], idxb[buf], isem[buf])

    def wait_idx(buf):
        pltpu.make_async_copy(
            xT_hbm.at[0, pl.ds(0, UNIT)], idxb[buf], isem[buf]).wait()

    def fire_gather(ibuf, g, b):
        pltpu.async_copy(
            table_hbm.at[idxb[ibuf].at[pl.ds(g * CH, CH)]], gbuf[b], gsem[b])

    def transpose_chunk(b, p, g):
        def blk(jj, _):
            j0 = jj * 16
            for m in range(4):
                sb = z16 + (j0 * HID + 16 * m)
                db = z16 + (g * ST_C + m * 2 * ST_HT + j0)
                for dd in range(16):
                    val = plsc.load_gather(gbuf[b], [z16, srcc[dd] + sb])
                    plsc.store_scatter(
                        tbuf[p], [z16, z16, z16, dstc[dd] + db], val * SCALE)
            return 0

        lax.fori_loop(0, CH // 16, blk, 0)

    def fire_out(u_local, p):
        uu = u0 + u_local
        s = uu // UNITS_PER_S
        q = uu % UNITS_PER_S
        for ht in range(8):
            pltpu.async_copy(
                tbuf[p].at[ht, pl.ds(0, 2), :, pl.ds(0, CH)],
                o_hbm.at[s, ht, pl.ds(2 * q, 2)],
                osem[p],
            )

    def wait_out(p):
        for d in _out_descs(tbuf[p], o_hbm, osem[p]):
            d.wait()

    for uu in range(4):
        fire_idx(uu, uu)
    for uu in range(3):
        wait_idx(uu)
        for g in range(2):
            fire_gather(uu, g, 2 * uu + g)

    def block(t, _):
        for k in range(4):
            u_rel = 4 * t + k
            p = k % 2
            for g in range(2):
                pos = 2 * k + g
                if g == 0:
                    @pl.when(t < NT - 1)
                    def _():
                        fire_idx(u_rel + 4, k)
                if pos < 2:
                    if g == 0:
                        wait_idx(3)
                    fire_gather(3, g, pos + 6)
                else:
                    @pl.when(t < NT - 1)
                    def _():
                        if g == 0:
                            wait_idx(k - 1)
                        fire_gather(k - 1, g, pos - 2)

                if g == 0:
                    if k < 2:
                        @pl.when(t >= 1)
                        def _():
                            wait_out(p)
                    else:
                        wait_out(p)

                _gather_desc(table_hbm, idxb[0], gbuf[pos], gsem[pos]).wait()
                transpose_chunk(pos, p, g)

            fire_out(u_rel, p)
        return 0

    lax.fori_loop(0, NT, block, 0)

    wait_out(0)
    wait_out(1)


@jax.jit
def kernel(x, emb_weight):
    xT = jnp.transpose(x).astype(jnp.int32)
    mesh = plsc.VectorSubcoreMesh(core_axis_name="c", subcore_axis_name="s")
    O = pl.kernel(
        _emb_body,
        out_type=jax.ShapeDtypeStruct((S, 8, B // CH, 8, CH), jnp.float32),
        mesh=mesh,
        compiler_params=pltpu.CompilerParams(
            use_tc_tiling_on_sc=False,
            needs_layout_passes=False,
            disable_bounds_checks=True,
        ),
        scratch_types=dict(
            idxb=[pltpu.VMEM((UNIT,), jnp.int32) for _ in range(4)],
            gbuf=[pltpu.VMEM((CH, HID), jnp.float32) for _ in range(8)],
            tbuf=[pltpu.VMEM((8, TB_C, 8, TB_L), jnp.float32) for _ in range(2)],
            isem=[pltpu.SemaphoreType.DMA for _ in range(4)],
            gsem=[pltpu.SemaphoreType.DMA for _ in range(8)],
            osem=[pltpu.SemaphoreType.DMA for _ in range(2)],
        ),
    )(xT, emb_weight)
    return O.transpose(2, 4, 0, 1, 3).reshape(B, S, HID)


# speedup vs baseline: 1.0441x; 1.0441x over previous
import functools

import jax
import jax.numpy as jnp
from jax import lax
from jax.experimental import pallas as pl
from jax.experimental.pallas import tpu as pltpu
from jax.experimental.pallas import tpu_sc as plsc

B = 4096
S = 200
HID = 64
NC, NS = 2, 16
NW = NC * NS
UNIT = 256
CH = 128
UNITS_PER_S = B // UNIT
NUNIT = S * UNITS_PER_S
UPW = NUNIT // NW
NT = UPW // 4
SCALE = 8.0
TB_C, TB_L = 2, 128
ST_HT, ST_C, ST_R = 8 * TB_C * TB_L, 8 * TB_L, TB_L


def _gather_desc(table_hbm, idxb, gbuf, gsem):
    return pltpu.make_async_copy(
        table_hbm.at[idxb.at[pl.ds(0, CH)]], gbuf, gsem)


def _out_descs(tbuf, o_hbm, osem):
    return [
        pltpu.make_async_copy(
            tbuf.at[ht],
            o_hbm.at[0, ht, pl.ds(0, 2)],
            osem,
        )
        for ht in range(8)
    ]


def _emb_body(xT_hbm, table_hbm, o_hbm, idxb, gbuf, tbuf, isem, gsem, osem):
    wid = lax.axis_index("s") * NC + lax.axis_index("c")
    u0 = wid * UPW

    iota = lax.iota(jnp.int32, 16)
    z16 = jnp.zeros((16,), jnp.int32)
    srcc, dstc = [], []
    for dd in range(16):
        hmod = lax.bitwise_and(iota + dd, 15)
        srcc.append(iota * HID + hmod)
        dstc.append(
            ST_HT * lax.shift_right_logical(hmod, 3)
            + ST_R * lax.bitwise_and(hmod, 7)
            + iota
        )

    def fire_idx(u_local, buf):
        uu = u0 + u_local
        s = uu // UNITS_PER_S
        q = uu % UNITS_PER_S
        pltpu.async_copy(
            xT_hbm.at[s, pl.ds(q * UNIT, UNIT)], idxb[buf], isem[buf])

    def wait_idx(buf):
        pltpu.make_async_copy(
            xT_hbm.at[0, pl.ds(0, UNIT)], idxb[buf], isem[buf]).wait()

    def fire_gather(ibuf, g, b):
        pltpu.async_copy(
            table_hbm.at[idxb[ibuf].at[pl.ds(g * CH, CH)]], gbuf[b], gsem[b])

    def transpose_chunk(b, p, g):
        def blk(jj, _):
            j0 = jj * 16
            for m in range(4):
                sb = z16 + (j0 * HID + 16 * m)
                db = z16 + (g * ST_C + m * 2 * ST_HT + j0)
                for dd in range(16):
                    val = plsc.load_gather(gbuf[b], [z16, srcc[dd] + sb])
                    plsc.store_scatter(
                        tbuf[p], [z16, z16, z16, dstc[dd] + db], val * SCALE)
            return 0

        lax.fori_loop(0, CH // 16, blk, 0)

    def fire_out(u_local, p):
        uu = u0 + u_local
        s = uu // UNITS_PER_S
        q = uu % UNITS_PER_S
        for ht in range(8):
            pltpu.async_copy(
                tbuf[p].at[ht],
                o_hbm.at[s, ht, pl.ds(2 * q, 2)],
                osem[p],
            )

    def wait_out(p):
        for d in _out_descs(tbuf[p], o_hbm, osem[p]):
            d.wait()

    for uu in range(4):
        fire_idx(uu, uu)
    for uu in range(3):
        wait_idx(uu)
        for g in range(2):
            fire_gather(uu, g, 2 * uu + g)

    def block(t, _):
        for k in range(4):
            u_rel = 4 * t + k
            p = k % 2
            for g in range(2):
                pos = 2 * k + g
                if g == 0:
                    @pl.when(t < NT - 1)
                    def _():
                        fire_idx(u_rel + 4, k)
                if pos < 2:
                    if g == 0:
                        wait_idx(3)
                    fire_gather(3, g, pos + 6)
                else:
                    @pl.when(t < NT - 1)
                    def _():
                        if g == 0:
                            wait_idx(k - 1)
                        fire_gather(k - 1, g, pos - 2)

                if g == 0:
                    if k < 2:
                        @pl.when(t >= 1)
                        def _():
                            wait_out(p)
                    else:
                        wait_out(p)

                _gather_desc(table_hbm, idxb[0], gbuf[pos], gsem[pos]).wait()
                transpose_chunk(pos, p, g)

            fire_out(u_rel, p)
        return 0

    lax.fori_loop(0, NT, block, 0)

    wait_out(0)
    wait_out(1)


@jax.jit
def kernel(x, emb_weight):
    xT = jnp.transpose(x).astype(jnp.int32)
    mesh = plsc.VectorSubcoreMesh(core_axis_name="c", subcore_axis_name="s")
    O = pl.kernel(
        _emb_body,
        out_type=jax.ShapeDtypeStruct((S, 8, B // CH, 8, CH), jnp.float32),
        mesh=mesh,
        compiler_params=pltpu.CompilerParams(
            use_tc_tiling_on_sc=False,
            needs_layout_passes=False,
            disable_bounds_checks=True,
        ),
        scratch_types=dict(
            idxb=[pltpu.VMEM((UNIT,), jnp.int32) for _ in range(4)],
            gbuf=[pltpu.VMEM((CH, HID), jnp.float32) for _ in range(8)],
            tbuf=[pltpu.VMEM((8, TB_C, 8, TB_L), jnp.float32) for _ in range(2)],
            isem=[pltpu.SemaphoreType.DMA for _ in range(4)],
            gsem=[pltpu.SemaphoreType.DMA for _ in range(8)],
            osem=[pltpu.SemaphoreType.DMA for _ in range(2)],
        ),
    )(xT, emb_weight)
    return O.transpose(2, 4, 0, 1, 3).reshape(B, S, HID)


# speedup vs baseline: 1.0487x; 1.0044x over previous
import functools

import jax
import jax.numpy as jnp
from jax import lax
from jax.experimental import pallas as pl
from jax.experimental.pallas import tpu as pltpu
from jax.experimental.pallas import tpu_sc as plsc

B = 4096
S = 200
HID = 64
NC, NS = 2, 16
NW = NC * NS
UNIT = 256
CH = 256
UNITS_PER_S = B // UNIT
NUNIT = S * UNITS_PER_S
UPW = NUNIT // NW
NT = UPW // 4
SCALE = 8.0
TB_C, TB_L = 2, 128
ST_HT, ST_C, ST_R = 8 * TB_C * TB_L, 8 * TB_L, TB_L


def _gather_desc(table_hbm, idxb, gbuf, gsem):
    return pltpu.make_async_copy(
        table_hbm.at[idxb.at[pl.ds(0, CH)]], gbuf, gsem)


def _out_descs(tbuf, o_hbm, osem):
    return [
        pltpu.make_async_copy(
            tbuf.at[ht, :, :, :],
            o_hbm.at[0, ht, pl.ds(0, 2), :, :],
            osem,
        )
        for ht in range(8)
    ]


def _emb_body(xT_hbm, table_hbm, o_hbm, idxb, gbuf, tbuf, isem, gsem, osem):
    wid = lax.axis_index("s") * NC + lax.axis_index("c")
    u0 = wid * UPW

    iota = lax.iota(jnp.int32, 16)
    z16 = jnp.zeros((16,), jnp.int32)
    srcc, dstc = [], []
    for dd in range(16):
        hmod = lax.bitwise_and(iota + dd, 15)
        srcc.append(iota * HID + hmod)
        dstc.append(
            ST_HT * lax.shift_right_logical(hmod, 3)
            + ST_R * lax.bitwise_and(hmod, 7)
            + iota
        )

    def fire_idx(u_local, buf):
        uu = u0 + u_local
        s = uu // UNITS_PER_S
        q = uu % UNITS_PER_S
        pltpu.async_copy(
            xT_hbm.at[s, pl.ds(q * UNIT, UNIT)], idxb[buf], isem[buf])

    def wait_idx(buf):
        pltpu.make_async_copy(
            xT_hbm.at[0, pl.ds(0, UNIT)], idxb[buf], isem[buf]).wait()

    def fire_gather(ibuf, b):
        pltpu.async_copy(table_hbm.at[idxb[ibuf]], gbuf[b], gsem[b])

    def transpose_chunk(b, p):
        def blk(jj, _):
            j0 = jj * 16
            c = lax.shift_right_logical(jj, 3)
            dboff = c * (ST_C - CH // 2) + j0
            for m in range(4):
                sb = z16 + (j0 * HID + 16 * m)
                db = z16 + (dboff + m * 2 * ST_HT)
                for dd in range(16):
                    val = plsc.load_gather(gbuf[b], [z16, srcc[dd] + sb])
                    plsc.store_scatter(
                        tbuf[p], [z16, z16, z16, dstc[dd] + db], val * SCALE)
            return 0

        lax.fori_loop(0, CH // 16, blk, 0)

    def fire_out(u_local, p):
        uu = u0 + u_local
        s = uu // UNITS_PER_S
        q = uu % UNITS_PER_S
        for ht in range(8):
            pltpu.async_copy(
                tbuf[p].at[ht, :, :, :],
                o_hbm.at[s, ht, pl.ds(2 * q, 2), :, :],
                osem[p],
            )

    def wait_out(p):
        for d in _out_descs(tbuf[p], o_hbm, osem[p]):
            d.wait()

    for uu in range(4):
        fire_idx(uu, uu)
    for uu in range(2):
        wait_idx(uu)
        fire_gather(uu, uu)

    def block(t, _):
        for k in range(4):
            u_rel = 4 * t + k
            p = k % 2
            if k < 2:
                wait_idx(k + 2)
                fire_gather(k + 2, k + 2)
            else:
                @pl.when(t < NT - 1)
                def _():
                    wait_idx(k - 2)
                    fire_gather(k - 2, k - 2)

            if k < 2:
                @pl.when(t >= 1)
                def _():
                    wait_out(p)
            else:
                wait_out(p)

            _gather_desc(table_hbm, idxb[0], gbuf[k], gsem[k]).wait()

            @pl.when(t < NT - 1)
            def _():
                fire_idx(u_rel + 4, k)

            transpose_chunk(k, p)
            fire_out(u_rel, p)
        return 0

    lax.fori_loop(0, NT, block, 0)

    wait_out(0)
    wait_out(1)


@jax.jit
def kernel(x, emb_weight):
    xT = jnp.transpose(x).astype(jnp.int32)
    mesh = plsc.VectorSubcoreMesh(core_axis_name="c", subcore_axis_name="s")
    O = pl.kernel(
        _emb_body,
        out_type=jax.ShapeDtypeStruct((S, 8, B // TB_L, 8, TB_L), jnp.float32),
        mesh=mesh,
        compiler_params=pltpu.CompilerParams(
            use_tc_tiling_on_sc=False,
            needs_layout_passes=False,
            disable_bounds_checks=True,
        ),
        scratch_types=dict(
            idxb=[pltpu.VMEM((UNIT,), jnp.int32) for _ in range(4)],
            gbuf=[pltpu.VMEM((CH, HID), jnp.float32) for _ in range(4)],
            tbuf=[pltpu.VMEM((8, TB_C, 8, TB_L), jnp.float32) for _ in range(2)],
            isem=[pltpu.SemaphoreType.DMA for _ in range(4)],
            gsem=[pltpu.SemaphoreType.DMA for _ in range(4)],
            osem=[pltpu.SemaphoreType.DMA for _ in range(2)],
        ),
    )(xT, emb_weight)
    return O.transpose(2, 4, 0, 1, 3).reshape(B, S, HID)
